# baseline (speedup 1.0000x reference)
import functools

import jax
import jax.numpy as jnp
from jax import lax
from jax.experimental import pallas as pl
from jax.experimental.pallas import tpu as pltpu
from jax.experimental.pallas import tpu_sc as plsc

N = 10000
CH = 128
K = 64
KS = 4
E = 160000
NW = 32
EPC = 16
EPW = 5120
EPAD = NW * EPW
CPW = EPW // EPC
ROWS = EPC * 8
BN = 400
NB = N // BN


def _edge_body(ei_ref, ea_ref, w1_ref, b1_ref, w2_ref, b2_ref, b_ref, g_ref):
    ea = ea_ref[...]
    h = jnp.dot(ea, w1_ref[...], preferred_element_type=jnp.float32)
    h = jnp.maximum(h + b1_ref[...], 0.0)
    p = jnp.dot(h, w2_ref[...], preferred_element_type=jnp.float32)
    p = jax.nn.sigmoid(p + b2_ref[...])
    v = p[:, 0:3] * (KS - 1.0)
    bot = jnp.floor(v)
    frac = v - bot
    bot_i = bot.astype(jnp.int32)
    src = ei_ref[...][:, 0:1]
    bs, gs = [], []
    for s in range(8):
        b = jnp.ones_like(src, dtype=jnp.float32)
        wi = jnp.zeros_like(src)
        for d in range(3):
            o = (s >> d) & 1
            fd = frac[:, d:d + 1]
            b = b * (fd if o == 1 else (1.0 - fd))
            idx = jnp.clip(bot_i[:, d:d + 1] + o, 0, KS - 1)
            wi = wi + idx * (KS ** d)
        bs.append(b)
        gs.append(src * K + wi)
    b_ref[...] = jnp.concatenate(bs, axis=1)
    g_ref[...] = jnp.concatenate(gs, axis=1)


def _edge_prep(eiT, edge_attr, w1p, b1p, w2p, b2p):
    EB = 2000
    grid = (E // EB,)
    return pl.pallas_call(
        _edge_body,
        grid=grid,
        in_specs=[
            pl.BlockSpec((EB, 2), lambda i: (i, 0)),
            pl.BlockSpec((EB, 16), lambda i: (i, 0)),
            pl.BlockSpec((16, 8), lambda i: (0, 0)),
            pl.BlockSpec((1, 8), lambda i: (0, 0)),
            pl.BlockSpec((8, 8), lambda i: (0, 0)),
            pl.BlockSpec((1, 8), lambda i: (0, 0)),
        ],
        out_specs=[
            pl.BlockSpec((EB, 8), lambda i: (i, 0)),
            pl.BlockSpec((EB, 8), lambda i: (i, 0)),
        ],
        out_shape=[
            jax.ShapeDtypeStruct((E, 8), jnp.float32),
            jax.ShapeDtypeStruct((E, 8), jnp.int32),
        ],
    )(eiT, edge_attr, w1p, b1p, w2p, b2p)


def _xw_body(x_ref, w_ref, o_ref):
    o_ref[...] = jnp.dot(x_ref[...], w_ref[0],
                         preferred_element_type=jnp.float32)


def _xw(x, W):
    out = pl.pallas_call(
        _xw_body,
        grid=(NB, K),
        in_specs=[
            pl.BlockSpec((BN, CH), lambda i, j: (i, 0)),
            pl.BlockSpec((1, CH, CH), lambda i, j: (j, 0, 0)),
        ],
        out_specs=pl.BlockSpec((BN, CH), lambda i, j: (i, j)),
        out_shape=jax.ShapeDtypeStruct((N, K * CH), jnp.float32),
    )(x, W)
    return out.reshape(N * K, CH)


SEG = 8
NSEG = CPW // SEG
NAGG = 10240


@functools.lru_cache(maxsize=None)
def _build_spline_sc():
    mesh = plsc.VectorSubcoreMesh(core_axis_name="c", subcore_axis_name="s")

    @functools.partial(
        pl.kernel,
        out_type=jax.ShapeDtypeStruct((2, N, CH), jnp.float32),
        mesh=mesh,
        scratch_types=[
            pltpu.VMEM((2, SEG, ROWS), jnp.int32),
            pltpu.VMEM((2, SEG, ROWS), jnp.float32),
            pltpu.VMEM((2, SEG, EPC), jnp.int32),
            pltpu.VMEM((2, ROWS, CH), jnp.float32),
            pltpu.VMEM((EPC, CH), jnp.float32),
            pltpu.VMEM((8, CH), jnp.float32),
            pltpu.VMEM_SHARED((NAGG, CH), jnp.float32),
            pltpu.SemaphoreType.DMA,
            pltpu.SemaphoreType.DMA,
            pltpu.SemaphoreType.DMA,
            pltpu.SemaphoreType.DMA,
        ],
    )
    def _spline_sc(xw_hbm, gi_hbm, bw_hbm, dl_hbm, out_hbm,
                   gi_v, bw_v, dl_v, rows_v, msg_v, zb_v, agg_sh,
                   gsem0, gsem1, msem0, msem1):
        c = lax.axis_index("c")
        t = lax.axis_index("s")
        w = c * 16 + t
        gsems = (gsem0, gsem1)
        msems = (msem0, msem1)
        zero = jnp.zeros((16,), jnp.float32)

        for i in range(8):
            for f in range(CH // 16):
                zb_v[i, pl.ds(f * 16, 16)] = zero
        base = t * 640
        for q in range(80):
            pltpu.sync_copy(zb_v, agg_sh.at[pl.ds(base + q * 8, 8)])
        plsc.subcore_barrier()

        def meta_copies(seg, mb):
            sl = pl.ds(seg * SEG, SEG)
            sem = msems[mb]
            return (
                pltpu.make_async_copy(gi_hbm.at[w, sl], gi_v.at[mb], sem),
                pltpu.make_async_copy(bw_hbm.at[w, sl], bw_v.at[mb], sem),
                pltpu.make_async_copy(dl_hbm.at[w, sl], dl_v.at[mb], sem),
            )

        def load_meta(seg, mb):
            for cp in meta_copies(seg, mb):
                cp.start()

        def wait_meta(seg, mb):
            for cp in meta_copies(seg, mb):
                cp.wait()

        def gather_copy(mb, cc, buf):
            return pltpu.make_async_copy(xw_hbm.at[gi_v.at[mb, cc]],
                                         rows_v.at[buf], gsems[buf])

        def process_chunk(mb, cc, buf):
            def pair_body(i2, carry):
                bv = bw_v[mb, cc, pl.ds(i2 * 16, 16)]
                for half in range(2):
                    for f in range(CH // 16):
                        acc = zero
                        for s8 in range(8):
                            r = i2 * 16 + half * 8 + s8
                            acc = acc + (bv[half * 8 + s8] *
                                         rows_v[buf, r, pl.ds(f * 16, 16)])
                        msg_v[2 * i2 + half, pl.ds(f * 16, 16)] = acc
                return carry
            lax.fori_loop(0, EPC // 2, pair_body, 0)
            pltpu.sync_copy(msg_v, agg_sh.at[dl_v.at[mb, cc]], add=True)

        def do_segment(seg, mb, nseg, nmb, has_next):
            def cpair(q, carry):
                cc0 = 2 * q
                gather_copy(mb, cc0 + 1, 1).start()
                gather_copy(mb, cc0, 0).wait()
                process_chunk(mb, cc0, 0)

                @pl.when(q < SEG // 2 - 1)
                def _():
                    gather_copy(mb, cc0 + 2, 0).start()

                @pl.when((q == SEG // 2 - 1) & has_next)
                def _():
                    wait_meta(nseg, nmb)
                    gather_copy(nmb, 0, 0).start()

                gather_copy(mb, cc0 + 1, 1).wait()
                process_chunk(mb, cc0 + 1, 1)
                return carry
            lax.fori_loop(0, SEG // 2, cpair, 0)

        load_meta(0, 0)
        load_meta(1, 1)
        wait_meta(0, 0)
        gather_copy(0, 0, 0).start()

        def seg_pair(p, carry):
            s0 = 2 * p
            do_segment(s0, 0, s0 + 1, 1, True)

            @pl.when(p < NSEG // 2 - 1)
            def _():
                load_meta(s0 + 2, 0)

            do_segment(s0 + 1, 1, s0 + 2, 0, p < NSEG // 2 - 1)

            @pl.when(p < NSEG // 2 - 1)
            def _():
                load_meta(s0 + 3, 1)
            return carry

        lax.fori_loop(0, NSEG // 2, seg_pair, 0)

        plsc.subcore_barrier()
        @pl.when(t < 15)
        def _():
            pltpu.sync_copy(agg_sh.at[pl.ds(t * 632, 632)],
                            out_hbm.at[c, pl.ds(t * 632, 632)])

        @pl.when(t == 15)
        def _():
            pltpu.sync_copy(agg_sh.at[pl.ds(9480, 520)],
                            out_hbm.at[c, pl.ds(9480, 520)])

    return _spline_sc


def _combine_body(do_relu, ag_ref, x_ref, r_ref, b_ref, o_ref):
    s = ag_ref[0] + ag_ref[1] + b_ref[...]
    s = s + jnp.dot(x_ref[...], r_ref[...], preferred_element_type=jnp.float32)
    o_ref[...] = jnp.maximum(s, 0.0) if do_relu else s


def _combine(aggp, xin, root, bias2d, do_relu):
    return pl.pallas_call(
        functools.partial(_combine_body, do_relu),
        grid=(NB,),
        in_specs=[
            pl.BlockSpec((2, BN, CH), lambda i: (0, i, 0)),
            pl.BlockSpec((BN, CH), lambda i: (i, 0)),
            pl.BlockSpec((CH, CH), lambda i: (0, 0)),
            pl.BlockSpec((1, CH), lambda i: (0, 0)),
        ],
        out_specs=pl.BlockSpec((BN, CH), lambda i: (i, 0)),
        out_shape=jax.ShapeDtypeStruct((N, CH), jnp.float32),
    )(aggp, xin, root, bias2d)


def kernel(x, edge_index, edge_attr, ep_w1, ep_b1, ep_w2, ep_b2,
           conv0_W, conv0_root, conv0_bias, conv1_W, conv1_root, conv1_bias):
    eiT = edge_index.astype(jnp.int32).T
    w1p = jnp.zeros((16, 8), jnp.float32).at[:, :6].set(ep_w1)
    b1p = jnp.zeros((1, 8), jnp.float32).at[0, :6].set(ep_b1)
    w2p = jnp.zeros((8, 8), jnp.float32).at[:6, :3].set(ep_w2)
    b2p = jnp.zeros((1, 8), jnp.float32).at[0, :3].set(ep_b2)

    b8, g8 = _edge_prep(eiT, edge_attr, w1p, b1p, w2p, b2p)
    pad = EPAD - E
    bw = jnp.pad(b8, ((0, pad), (0, 0))).reshape(NW, CPW, ROWS)
    gi = jnp.pad(g8, ((0, pad), (0, 0))).reshape(NW, CPW, ROWS)
    dl = jnp.pad(edge_index[1].astype(jnp.int32), (0, pad))
    dl = dl.reshape(NW, CPW, EPC)

    bias0 = conv0_bias.reshape(1, CH)
    bias1 = conv1_bias.reshape(1, CH)

    spline_sc = _build_spline_sc()

    xw0 = _xw(x, conv0_W)
    ag0 = spline_sc(xw0, gi, bw, dl)
    h1 = _combine(ag0, x, conv0_root, bias0, True)

    xw1 = _xw(h1, conv1_W)
    ag1 = spline_sc(xw1, gi, bw, dl)
    out = _combine(ag1, h1, conv1_root, bias1, False)
    return out


# speedup vs baseline: 1.3714x; 1.3714x over previous
import functools

import jax
import jax.numpy as jnp
from jax import lax
from jax.experimental import pallas as pl
from jax.experimental.pallas import tpu as pltpu
from jax.experimental.pallas import tpu_sc as plsc

N = 10000
CH = 128
K = 64
KS = 4
E = 160000
NW = 32
EPC = 16
EPW = 5120
EPAD = NW * EPW
CPW = EPW // EPC
ROWS = EPC * 8
BN = 400
NB = N // BN


def _edge_body(ei_ref, ea_ref, w1_ref, b1_ref, w2_ref, b2_ref, b_ref, g_ref):
    ea = ea_ref[...]
    h = jnp.dot(ea, w1_ref[...], preferred_element_type=jnp.float32)
    h = jnp.maximum(h + b1_ref[...], 0.0)
    p = jnp.dot(h, w2_ref[...], preferred_element_type=jnp.float32)
    p = jax.nn.sigmoid(p + b2_ref[...])
    v = p[:, 0:3] * (KS - 1.0)
    bot = jnp.floor(v)
    frac = v - bot
    bot_i = bot.astype(jnp.int32)
    src = ei_ref[...][:, 0:1]
    bs, gs = [], []
    for s in range(8):
        b = jnp.ones_like(src, dtype=jnp.float32)
        wi = jnp.zeros_like(src)
        for d in range(3):
            o = (s >> d) & 1
            fd = frac[:, d:d + 1]
            b = b * (fd if o == 1 else (1.0 - fd))
            idx = jnp.clip(bot_i[:, d:d + 1] + o, 0, KS - 1)
            wi = wi + idx * (KS ** d)
        bs.append(b)
        gs.append(src * K + wi)
    b_ref[...] = jnp.concatenate(bs, axis=1)
    g_ref[...] = jnp.concatenate(gs, axis=1)


def _edge_prep(eiT, edge_attr, w1p, b1p, w2p, b2p):
    EB = 2000
    grid = (E // EB,)
    return pl.pallas_call(
        _edge_body,
        grid=grid,
        in_specs=[
            pl.BlockSpec((EB, 2), lambda i: (i, 0)),
            pl.BlockSpec((EB, 16), lambda i: (i, 0)),
            pl.BlockSpec((16, 8), lambda i: (0, 0)),
            pl.BlockSpec((1, 8), lambda i: (0, 0)),
            pl.BlockSpec((8, 8), lambda i: (0, 0)),
            pl.BlockSpec((1, 8), lambda i: (0, 0)),
        ],
        out_specs=[
            pl.BlockSpec((EB, 8), lambda i: (i, 0)),
            pl.BlockSpec((EB, 8), lambda i: (i, 0)),
        ],
        out_shape=[
            jax.ShapeDtypeStruct((E, 8), jnp.float32),
            jax.ShapeDtypeStruct((E, 8), jnp.int32),
        ],
    )(eiT, edge_attr, w1p, b1p, w2p, b2p)


KB = 8


def _xw_body(x_ref, w_ref, o_ref):
    x = x_ref[...]
    for kk in range(KB):
        o_ref[:, kk, :] = jnp.dot(x, w_ref[kk],
                                  preferred_element_type=jnp.float32)


def _xw(x, W):
    out = pl.pallas_call(
        _xw_body,
        grid=(NB, K // KB),
        in_specs=[
            pl.BlockSpec((BN, CH), lambda i, j: (i, 0)),
            pl.BlockSpec((KB, CH, CH), lambda i, j: (j, 0, 0)),
        ],
        out_specs=pl.BlockSpec((BN, KB, CH), lambda i, j: (i, j, 0)),
        out_shape=jax.ShapeDtypeStruct((N, K, CH), jnp.float32),
    )(x, W)
    return out.reshape(N * K, CH)


SEG = 8
NSEG = CPW // SEG
NAGG = 10240


@functools.lru_cache(maxsize=None)
def _build_spline_sc():
    mesh = plsc.VectorSubcoreMesh(core_axis_name="c", subcore_axis_name="s")

    @functools.partial(
        pl.kernel,
        out_type=jax.ShapeDtypeStruct((2, N, CH), jnp.float32),
        mesh=mesh,
        scratch_types=[
            pltpu.VMEM((2, SEG, ROWS), jnp.int32),
            pltpu.VMEM((2, SEG, ROWS), jnp.float32),
            pltpu.VMEM((2, SEG, EPC), jnp.int32),
            pltpu.VMEM((2, ROWS, CH), jnp.float32),
            pltpu.VMEM((EPC, CH), jnp.float32),
            pltpu.VMEM((32, CH), jnp.float32),
            pltpu.VMEM_SHARED((NAGG, CH), jnp.float32),
            pltpu.SemaphoreType.DMA,
            pltpu.SemaphoreType.DMA,
            pltpu.SemaphoreType.DMA,
            pltpu.SemaphoreType.DMA,
        ],
    )
    def _spline_sc(xw_hbm, gi_hbm, bw_hbm, dl_hbm, out_hbm,
                   gi_v, bw_v, dl_v, rows_v, msg_v, zb_v, agg_sh,
                   gsem0, gsem1, msem0, msem1):
        c = lax.axis_index("c")
        t = lax.axis_index("s")
        w = c * 16 + t
        gsems = (gsem0, gsem1)
        msems = (msem0, msem1)
        zero = jnp.zeros((16,), jnp.float32)

        for i in range(32):
            for f in range(CH // 16):
                zb_v[i, pl.ds(f * 16, 16)] = zero
        base = t * 640
        for q in range(20):
            pltpu.sync_copy(zb_v, agg_sh.at[pl.ds(base + q * 32, 32)])
        plsc.subcore_barrier()

        def meta_copies(seg, mb):
            sl = pl.ds(seg * SEG, SEG)
            sem = msems[mb]
            return (
                pltpu.make_async_copy(gi_hbm.at[w, sl], gi_v.at[mb], sem),
                pltpu.make_async_copy(bw_hbm.at[w, sl], bw_v.at[mb], sem),
                pltpu.make_async_copy(dl_hbm.at[w, sl], dl_v.at[mb], sem),
            )

        def load_meta(seg, mb):
            for cp in meta_copies(seg, mb):
                cp.start()

        def wait_meta(seg, mb):
            for cp in meta_copies(seg, mb):
                cp.wait()

        def gather_copy(mb, cc, buf):
            return pltpu.make_async_copy(xw_hbm.at[gi_v.at[mb, cc]],
                                         rows_v.at[buf], gsems[buf])

        def process_chunk(mb, cc, buf):
            def pair_body(i2, carry):
                bv = bw_v[mb, cc, pl.ds(i2 * 16, 16)]
                for half in range(2):
                    for f in range(CH // 16):
                        acc = zero
                        for s8 in range(8):
                            r = i2 * 16 + half * 8 + s8
                            acc = acc + (bv[half * 8 + s8] *
                                         rows_v[buf, r, pl.ds(f * 16, 16)])
                        msg_v[2 * i2 + half, pl.ds(f * 16, 16)] = acc
                return carry
            lax.fori_loop(0, EPC // 2, pair_body, 0)
            pltpu.sync_copy(msg_v, agg_sh.at[dl_v.at[mb, cc]], add=True)

        def do_segment(seg, mb, nseg, nmb, has_next):
            def cpair(q, carry):
                cc0 = 2 * q
                gather_copy(mb, cc0 + 1, 1).start()
                gather_copy(mb, cc0, 0).wait()
                process_chunk(mb, cc0, 0)

                @pl.when(q < SEG // 2 - 1)
                def _():
                    gather_copy(mb, cc0 + 2, 0).start()

                @pl.when((q == SEG // 2 - 1) & has_next)
                def _():
                    wait_meta(nseg, nmb)
                    gather_copy(nmb, 0, 0).start()

                gather_copy(mb, cc0 + 1, 1).wait()
                process_chunk(mb, cc0 + 1, 1)
                return carry
            lax.fori_loop(0, SEG // 2, cpair, 0)

        load_meta(0, 0)
        load_meta(1, 1)
        wait_meta(0, 0)
        gather_copy(0, 0, 0).start()

        def seg_pair(p, carry):
            s0 = 2 * p
            do_segment(s0, 0, s0 + 1, 1, True)

            @pl.when(p < NSEG // 2 - 1)
            def _():
                load_meta(s0 + 2, 0)

            do_segment(s0 + 1, 1, s0 + 2, 0, p < NSEG // 2 - 1)

            @pl.when(p < NSEG // 2 - 1)
            def _():
                load_meta(s0 + 3, 1)
            return carry

        lax.fori_loop(0, NSEG // 2, seg_pair, 0)

        plsc.subcore_barrier()
        @pl.when(t < 15)
        def _():
            pltpu.sync_copy(agg_sh.at[pl.ds(t * 632, 632)],
                            out_hbm.at[c, pl.ds(t * 632, 632)])

        @pl.when(t == 15)
        def _():
            pltpu.sync_copy(agg_sh.at[pl.ds(9480, 520)],
                            out_hbm.at[c, pl.ds(9480, 520)])

    return _spline_sc


def _combine_body(do_relu, ag_ref, x_ref, r_ref, b_ref, o_ref):
    s = ag_ref[0] + ag_ref[1] + b_ref[...]
    s = s + jnp.dot(x_ref[...], r_ref[...], preferred_element_type=jnp.float32)
    o_ref[...] = jnp.maximum(s, 0.0) if do_relu else s


def _combine(aggp, xin, root, bias2d, do_relu):
    return pl.pallas_call(
        functools.partial(_combine_body, do_relu),
        grid=(NB,),
        in_specs=[
            pl.BlockSpec((2, BN, CH), lambda i: (0, i, 0)),
            pl.BlockSpec((BN, CH), lambda i: (i, 0)),
            pl.BlockSpec((CH, CH), lambda i: (0, 0)),
            pl.BlockSpec((1, CH), lambda i: (0, 0)),
        ],
        out_specs=pl.BlockSpec((BN, CH), lambda i: (i, 0)),
        out_shape=jax.ShapeDtypeStruct((N, CH), jnp.float32),
    )(aggp, xin, root, bias2d)


def kernel(x, edge_index, edge_attr, ep_w1, ep_b1, ep_w2, ep_b2,
           conv0_W, conv0_root, conv0_bias, conv1_W, conv1_root, conv1_bias):
    eiT = edge_index.astype(jnp.int32).T
    w1p = jnp.zeros((16, 8), jnp.float32).at[:, :6].set(ep_w1)
    b1p = jnp.zeros((1, 8), jnp.float32).at[0, :6].set(ep_b1)
    w2p = jnp.zeros((8, 8), jnp.float32).at[:6, :3].set(ep_w2)
    b2p = jnp.zeros((1, 8), jnp.float32).at[0, :3].set(ep_b2)

    b8, g8 = _edge_prep(eiT, edge_attr, w1p, b1p, w2p, b2p)
    pad = EPAD - E
    bw = jnp.pad(b8, ((0, pad), (0, 0))).reshape(NW, CPW, ROWS)
    gi = jnp.pad(g8, ((0, pad), (0, 0))).reshape(NW, CPW, ROWS)
    dl = jnp.pad(edge_index[1].astype(jnp.int32), (0, pad))
    dl = dl.reshape(NW, CPW, EPC)

    bias0 = conv0_bias.reshape(1, CH)
    bias1 = conv1_bias.reshape(1, CH)

    spline_sc = _build_spline_sc()

    xw0 = _xw(x, conv0_W)
    ag0 = spline_sc(xw0, gi, bw, dl)
    h1 = _combine(ag0, x, conv0_root, bias0, True)

    xw1 = _xw(h1, conv1_W)
    ag1 = spline_sc(xw1, gi, bw, dl)
    out = _combine(ag1, h1, conv1_root, bias1, False)
    return out


# speedup vs baseline: 1.3737x; 1.0017x over previous
import functools

import jax
import jax.numpy as jnp
from jax import lax
from jax.experimental import pallas as pl
from jax.experimental.pallas import tpu as pltpu
from jax.experimental.pallas import tpu_sc as plsc

N = 10000
CH = 128
K = 64
KS = 4
E = 160000
NW = 32
EPC = 8
EPW = 5120
EPAD = NW * EPW
CPW = EPW // EPC
ROWS = EPC * 8
BN = 400
NB = N // BN


def _edge_body(ei_ref, ea_ref, w1_ref, b1_ref, w2_ref, b2_ref, b_ref, g_ref):
    ea = ea_ref[...]
    h = jnp.dot(ea, w1_ref[...], preferred_element_type=jnp.float32)
    h = jnp.maximum(h + b1_ref[...], 0.0)
    p = jnp.dot(h, w2_ref[...], preferred_element_type=jnp.float32)
    p = jax.nn.sigmoid(p + b2_ref[...])
    v = p[:, 0:3] * (KS - 1.0)
    bot = jnp.floor(v)
    frac = v - bot
    bot_i = bot.astype(jnp.int32)
    src = ei_ref[...][:, 0:1]
    bs, gs = [], []
    for s in range(8):
        b = jnp.ones_like(src, dtype=jnp.float32)
        wi = jnp.zeros_like(src)
        for d in range(3):
            o = (s >> d) & 1
            fd = frac[:, d:d + 1]
            b = b * (fd if o == 1 else (1.0 - fd))
            idx = jnp.clip(bot_i[:, d:d + 1] + o, 0, KS - 1)
            wi = wi + idx * (KS ** d)
        bs.append(b)
        gs.append(src * K + wi)
    b_ref[...] = jnp.concatenate(bs, axis=1)
    g_ref[...] = jnp.concatenate(gs, axis=1)


def _edge_prep(eiT, edge_attr, w1p, b1p, w2p, b2p):
    EB = 2000
    grid = (E // EB,)
    return pl.pallas_call(
        _edge_body,
        grid=grid,
        in_specs=[
            pl.BlockSpec((EB, 2), lambda i: (i, 0)),
            pl.BlockSpec((EB, 16), lambda i: (i, 0)),
            pl.BlockSpec((16, 8), lambda i: (0, 0)),
            pl.BlockSpec((1, 8), lambda i: (0, 0)),
            pl.BlockSpec((8, 8), lambda i: (0, 0)),
            pl.BlockSpec((1, 8), lambda i: (0, 0)),
        ],
        out_specs=[
            pl.BlockSpec((EB, 8), lambda i: (i, 0)),
            pl.BlockSpec((EB, 8), lambda i: (i, 0)),
        ],
        out_shape=[
            jax.ShapeDtypeStruct((E, 8), jnp.float32),
            jax.ShapeDtypeStruct((E, 8), jnp.int32),
        ],
    )(eiT, edge_attr, w1p, b1p, w2p, b2p)


KB = 8


def _xw_body(x_ref, w_ref, o_ref):
    x = x_ref[...]
    for kk in range(KB):
        o_ref[:, kk, :] = jnp.dot(x, w_ref[kk],
                                  preferred_element_type=jnp.float32)


def _xw(x, W):
    out = pl.pallas_call(
        _xw_body,
        grid=(NB, K // KB),
        in_specs=[
            pl.BlockSpec((BN, CH), lambda i, j: (i, 0)),
            pl.BlockSpec((KB, CH, CH), lambda i, j: (j, 0, 0)),
        ],
        out_specs=pl.BlockSpec((BN, KB, CH), lambda i, j: (i, j, 0)),
        out_shape=jax.ShapeDtypeStruct((N, K, CH), jnp.float32),
    )(x, W)
    return out.reshape(N * K, CH)


SEG = 16
NSEG = CPW // SEG
NAGG = 10112


@functools.lru_cache(maxsize=None)
def _build_spline_sc():
    mesh = plsc.VectorSubcoreMesh(core_axis_name="c", subcore_axis_name="s")

    @functools.partial(
        pl.kernel,
        out_type=jax.ShapeDtypeStruct((2, N, CH), jnp.float32),
        mesh=mesh,
        scratch_types=[
            pltpu.VMEM((2, SEG, ROWS), jnp.int32),
            pltpu.VMEM((2, SEG, ROWS), jnp.float32),
            pltpu.VMEM((2, SEG, EPC), jnp.int32),
            pltpu.VMEM((4, ROWS, CH), jnp.float32),
            pltpu.VMEM((2, EPC, CH), jnp.float32),
            pltpu.VMEM((8, CH), jnp.float32),
            pltpu.VMEM_SHARED((NAGG, CH), jnp.float32),
            [pltpu.SemaphoreType.DMA] * 4,
            [pltpu.SemaphoreType.DMA] * 2,
            [pltpu.SemaphoreType.DMA] * 2,
        ],
    )
    def _spline_sc(xw_hbm, gi_hbm, bw_hbm, dl_hbm, out_hbm,
                   gi_v, bw_v, dl_v, rows_v, msg_v, zb_v, agg_sh,
                   gsems, ssems, msems):
        c = lax.axis_index("c")
        t = lax.axis_index("s")
        w = c * 16 + t
        zero = jnp.zeros((16,), jnp.float32)

        for i in range(8):
            for f in range(CH // 16):
                zb_v[i, pl.ds(f * 16, 16)] = zero
        base = t * 632
        for q in range(79):
            pltpu.sync_copy(zb_v, agg_sh.at[pl.ds(base + q * 8, 8)])
        plsc.subcore_barrier()

        def meta_copies(seg, mb):
            sl = pl.ds(seg * SEG, SEG)
            sem = msems[mb]
            return (
                pltpu.make_async_copy(gi_hbm.at[w, sl], gi_v.at[mb], sem),
                pltpu.make_async_copy(bw_hbm.at[w, sl], bw_v.at[mb], sem),
                pltpu.make_async_copy(dl_hbm.at[w, sl], dl_v.at[mb], sem),
            )

        def load_meta(seg, mb):
            for cp in meta_copies(seg, mb):
                cp.start()

        def wait_meta(seg, mb):
            for cp in meta_copies(seg, mb):
                cp.wait()

        def gather_copy(mb, cc, buf):
            return pltpu.make_async_copy(xw_hbm.at[gi_v.at[mb, cc]],
                                         rows_v.at[buf], gsems[buf])

        def scatter_start(mb, cc, m):
            pltpu.async_copy(msg_v.at[m], agg_sh.at[dl_v.at[mb, cc]],
                             ssems[m], add=True)

        def scatter_wait(mb, cc, m):
            pltpu.make_async_copy(msg_v.at[m], agg_sh.at[dl_v.at[mb, cc]],
                                  ssems[m]).wait()

        def compute_chunk(mb, cc, buf, m):
            def pair_body(i2, carry):
                bv = bw_v[mb, cc, pl.ds(i2 * 16, 16)]
                for half in range(2):
                    for f in range(CH // 16):
                        acc = zero
                        for s8 in range(8):
                            r = i2 * 16 + half * 8 + s8
                            acc = acc + (bv[half * 8 + s8] *
                                         rows_v[buf, r, pl.ds(f * 16, 16)])
                        msg_v[m, 2 * i2 + half, pl.ds(f * 16, 16)] = acc
                return carry
            lax.fori_loop(0, EPC // 2, pair_body, 0)

        def do_segment(seg, mb, nseg, nmb, has_next):
            def cquad(q, carry):
                for b in range(4):
                    cc = 4 * q + b
                    if b == 0:
                        @pl.when(q < SEG // 4 - 1)
                        def _():
                            gather_copy(mb, cc + 3, (b + 3) % 4).start()

                        @pl.when(q == SEG // 4 - 1)
                        def _():
                            gather_copy(mb, SEG - 1, (b + 3) % 4).start()

                        @pl.when((q == SEG // 4 - 1) & has_next)
                        def _():
                            wait_meta(nseg, nmb)
                    else:
                        @pl.when(q < SEG // 4 - 1)
                        def _():
                            gather_copy(mb, cc + 3, (b + 3) % 4).start()

                        @pl.when((q == SEG // 4 - 1) & has_next)
                        def _():
                            gather_copy(nmb, b - 1, (b + 3) % 4).start()
                    gather_copy(mb, cc, b).wait()
                    m = b % 2
                    if b >= 2:
                        scatter_wait(mb, cc - 2, m)
                    else:
                        @pl.when(q > 0)
                        def _():
                            scatter_wait(mb, cc - 2, m)
                    compute_chunk(mb, cc, b, m)
                    scatter_start(mb, cc, m)
                return carry
            lax.fori_loop(0, SEG // 4, cquad, 0)
            scatter_wait(mb, SEG - 2, 0)
            scatter_wait(mb, SEG - 1, 1)

        load_meta(0, 0)
        load_meta(1, 1)
        wait_meta(0, 0)
        gather_copy(0, 0, 0).start()
        gather_copy(0, 1, 1).start()
        gather_copy(0, 2, 2).start()

        def seg_pair(p, carry):
            s0 = 2 * p
            do_segment(s0, 0, s0 + 1, 1, True)

            @pl.when(p < NSEG // 2 - 1)
            def _():
                load_meta(s0 + 2, 0)

            do_segment(s0 + 1, 1, s0 + 2, 0, p < NSEG // 2 - 1)

            @pl.when(p < NSEG // 2 - 1)
            def _():
                load_meta(s0 + 3, 1)
            return carry

        lax.fori_loop(0, NSEG // 2, seg_pair, 0)

        plsc.subcore_barrier()
        @pl.when(t < 15)
        def _():
            pltpu.sync_copy(agg_sh.at[pl.ds(t * 632, 632)],
                            out_hbm.at[c, pl.ds(t * 632, 632)])

        @pl.when(t == 15)
        def _():
            pltpu.sync_copy(agg_sh.at[pl.ds(9480, 520)],
                            out_hbm.at[c, pl.ds(9480, 520)])

    return _spline_sc


def _combine_body(do_relu, ag_ref, x_ref, r_ref, b_ref, o_ref):
    s = ag_ref[0] + ag_ref[1] + b_ref[...]
    s = s + jnp.dot(x_ref[...], r_ref[...], preferred_element_type=jnp.float32)
    o_ref[...] = jnp.maximum(s, 0.0) if do_relu else s


def _combine(aggp, xin, root, bias2d, do_relu):
    return pl.pallas_call(
        functools.partial(_combine_body, do_relu),
        grid=(NB,),
        in_specs=[
            pl.BlockSpec((2, BN, CH), lambda i: (0, i, 0)),
            pl.BlockSpec((BN, CH), lambda i: (i, 0)),
            pl.BlockSpec((CH, CH), lambda i: (0, 0)),
            pl.BlockSpec((1, CH), lambda i: (0, 0)),
        ],
        out_specs=pl.BlockSpec((BN, CH), lambda i: (i, 0)),
        out_shape=jax.ShapeDtypeStruct((N, CH), jnp.float32),
    )(aggp, xin, root, bias2d)


def kernel(x, edge_index, edge_attr, ep_w1, ep_b1, ep_w2, ep_b2,
           conv0_W, conv0_root, conv0_bias, conv1_W, conv1_root, conv1_bias):
    eiT = edge_index.astype(jnp.int32).T
    w1p = jnp.zeros((16, 8), jnp.float32).at[:, :6].set(ep_w1)
    b1p = jnp.zeros((1, 8), jnp.float32).at[0, :6].set(ep_b1)
    w2p = jnp.zeros((8, 8), jnp.float32).at[:6, :3].set(ep_w2)
    b2p = jnp.zeros((1, 8), jnp.float32).at[0, :3].set(ep_b2)

    b8, g8 = _edge_prep(eiT, edge_attr, w1p, b1p, w2p, b2p)
    pad = EPAD - E
    bw = jnp.pad(b8, ((0, pad), (0, 0))).reshape(NW, CPW, ROWS)
    gi = jnp.pad(g8, ((0, pad), (0, 0))).reshape(NW, CPW, ROWS)
    dl = jnp.pad(edge_index[1].astype(jnp.int32), (0, pad))
    dl = dl.reshape(NW, CPW, EPC)

    bias0 = conv0_bias.reshape(1, CH)
    bias1 = conv1_bias.reshape(1, CH)

    spline_sc = _build_spline_sc()

    xw0 = _xw(x, conv0_W)
    ag0 = spline_sc(xw0, gi, bw, dl)
    h1 = _combine(ag0, x, conv0_root, bias0, True)

    xw1 = _xw(h1, conv1_W)
    ag1 = spline_sc(xw1, gi, bw, dl)
    out = _combine(ag1, h1, conv1_root, bias1, False)
    return out
